# Initial kernel scaffold; baseline (speedup 1.0000x reference)
import jax
import jax.numpy as jnp
from jax.experimental import pallas as pl


def kernel(hidden_states, gate_w, Wg, Wu, Wd):
    raise NotImplementedError("write your pallas kernel here")


# speedup vs baseline: 2.9760x; 2.9760x over previous
import functools

import jax
import jax.numpy as jnp
from jax import lax
from jax.experimental import pallas as pl
from jax.experimental.pallas import tpu as pltpu
from jax.experimental.pallas import tpu_sc as plsc

TILE = 128

_SC_CORES = 2
_SC_SUBCORES = 16
_NW = _SC_CORES * _SC_SUBCORES
_SC_CHUNK = 64


def _router_body(x_ref, gw_ref, logits_ref, eid_ref):
    x = x_ref[...]
    logits = lax.dot_general(x, gw_ref[...], (((1,), (1,)), ((), ())),
                             preferred_element_type=jnp.float32)
    logits_ref[...] = logits
    eid_ref[...] = jnp.argmax(logits, axis=1, keepdims=True).astype(jnp.int32)


def _router(flat, gate_w):
    s, _ = flat.shape
    e, _ = gate_w.shape
    return pl.pallas_call(
        _router_body,
        out_shape=(
            jax.ShapeDtypeStruct((s, e), jnp.float32),
            jax.ShapeDtypeStruct((s, 1), jnp.int32),
        ),
    )(flat, gate_w)


def _ffn_body(e_ref, xs_ref, wg_ref, wu_ref, wd_ref, out_ref):
    x = xs_ref[...]
    dn = (((1,), (1,)), ((), ()))
    h1 = lax.dot_general(x, wg_ref[0], dn, preferred_element_type=jnp.float32)
    h2 = lax.dot_general(x, wu_ref[0], dn, preferred_element_type=jnp.float32)
    h = (h1 * jax.nn.sigmoid(h1)) * h2
    out_ref[...] = lax.dot_general(h, wd_ref[0], dn,
                                   preferred_element_type=jnp.float32)


def _grouped_ffn(x_slots, Wg, Wu, Wd, tile_e, nt):
    _, d = x_slots.shape
    _, i, _ = Wg.shape
    grid_spec = pltpu.PrefetchScalarGridSpec(
        num_scalar_prefetch=1,
        grid=(nt,),
        in_specs=[
            pl.BlockSpec((TILE, d), lambda t, e: (t, 0)),
            pl.BlockSpec((1, i, d), lambda t, e: (e[t], 0, 0)),
            pl.BlockSpec((1, i, d), lambda t, e: (e[t], 0, 0)),
            pl.BlockSpec((1, d, i), lambda t, e: (e[t], 0, 0)),
        ],
        out_specs=pl.BlockSpec((TILE, d), lambda t, e: (t, 0)),
    )
    return pl.pallas_call(
        _ffn_body,
        grid_spec=grid_spec,
        out_shape=jax.ShapeDtypeStruct((nt * TILE, d), jnp.float32),
        compiler_params=pltpu.CompilerParams(
            dimension_semantics=("arbitrary",)),
    )(tile_e, x_slots, Wg, Wu, Wd)


def _sc_gather_rows(table, idx):
    _, d = table.shape
    n = idx.shape[0]
    b_per_w = n // _NW
    chunk = min(b_per_w, _SC_CHUNK)
    n_chunks = b_per_w // chunk
    mesh = plsc.VectorSubcoreMesh(
        core_axis_name="c", subcore_axis_name="s",
        num_cores=_SC_CORES, num_subcores=_SC_SUBCORES)

    @functools.partial(
        pl.kernel,
        out_type=jax.ShapeDtypeStruct((n, d), jnp.float32),
        mesh=mesh,
        scratch_types=[
            pltpu.VMEM((chunk,), jnp.int32),
            pltpu.VMEM((chunk, d), jnp.float32),
            pltpu.SemaphoreType.DMA,
        ],
    )
    def k(table_hbm, idx_hbm, out_hbm, idx_v, rows_v, sem):
        wid = lax.axis_index("s") * _SC_CORES + lax.axis_index("c")
        base = wid * b_per_w
        for c in range(n_chunks):
            off = base + c * chunk
            pltpu.sync_copy(idx_hbm.at[pl.ds(off, chunk)], idx_v)
            pltpu.async_copy(table_hbm.at[idx_v], rows_v, sem).wait()
            pltpu.sync_copy(rows_v, out_hbm.at[pl.ds(off, chunk)])

    return k(table, idx)


def kernel(hidden_states, gate_w, Wg, Wu, Wd):
    bsz, seq_len, d = hidden_states.shape
    e = gate_w.shape[0]
    flat = hidden_states.reshape(-1, d)
    s = flat.shape[0]

    logits, eids = _router(flat, gate_w)
    eid = eids[:, 0]

    sorted_eid, perm = lax.sort((eid, jnp.arange(s, dtype=jnp.int32)),
                                dimension=0, num_keys=1, is_stable=True)
    inv_perm = jnp.zeros((s,), jnp.int32).at[perm].set(
        jnp.arange(s, dtype=jnp.int32))
    counts = jnp.bincount(eid, length=e).astype(jnp.int32)
    starts = jnp.concatenate(
        [jnp.zeros((1,), jnp.int32), jnp.cumsum(counts)[:-1].astype(jnp.int32)])
    tiles_per_e = (counts + TILE - 1) // TILE
    nt = s // TILE + e
    tile_e = jnp.repeat(jnp.arange(e, dtype=jnp.int32), tiles_per_e,
                        total_repeat_length=nt)
    tile_base = jnp.concatenate(
        [jnp.zeros((1,), jnp.int32),
         jnp.cumsum(tiles_per_e)[:-1].astype(jnp.int32)])
    tile_k = jnp.arange(nt, dtype=jnp.int32) - tile_base[tile_e]
    tile_start = starts[tile_e] + tile_k * TILE

    slot_ids = jnp.arange(nt * TILE, dtype=jnp.int32)
    slot_sorted_pos = jnp.clip(
        tile_start[slot_ids // TILE] + slot_ids % TILE, 0, s - 1)
    slot_token = perm[slot_sorted_pos]
    token_slot = (tile_base[sorted_eid] * TILE
                  + (jnp.arange(s, dtype=jnp.int32) - starts[sorted_eid]))
    token_slot = token_slot[inv_perm]

    x_slots = _sc_gather_rows(flat, slot_token)
    out_slots = _grouped_ffn(x_slots, Wg, Wu, Wd, tile_e, nt)
    out_flat = _sc_gather_rows(out_slots, token_slot)

    return out_flat.reshape(bsz, seq_len, d), logits


# speedup vs baseline: 4.4467x; 1.4942x over previous
import functools

import jax
import jax.numpy as jnp
from jax import lax
from jax.experimental import pallas as pl
from jax.experimental.pallas import tpu as pltpu
from jax.experimental.pallas import tpu_sc as plsc

TILE = 128
PAD_ROWS = 768

_SC_CORES = 2
_SC_SUBCORES = 16
_NW = _SC_CORES * _SC_SUBCORES
_SC_BUF_BYTES = 384 * 1024


def _router_body(x_ref, gw_ref, logits_ref, eid_ref):
    x = x_ref[...]
    logits = lax.dot_general(x, gw_ref[...], (((1,), (1,)), ((), ())),
                             preferred_element_type=jnp.float32)
    logits_ref[...] = logits
    eid_ref[...] = jnp.argmax(logits, axis=1, keepdims=True).astype(jnp.int32)


def _router(flat, gate_w):
    s, _ = flat.shape
    e, _ = gate_w.shape
    return pl.pallas_call(
        _router_body,
        out_shape=(
            jax.ShapeDtypeStruct((s, e), jnp.float32),
            jax.ShapeDtypeStruct((s, 1), jnp.int32),
        ),
    )(flat, gate_w)


def _ffn_body(e_ref, base_ref, xs_ref, wg_ref, wu_ref, wd_ref, out_ref):
    t = pl.program_id(0)
    base = pl.multiple_of(base_ref[t], 8)
    x = xs_ref[pl.ds(base, TILE), :]
    dn = (((1,), (1,)), ((), ()))
    h1 = lax.dot_general(x, wg_ref[0], dn, preferred_element_type=jnp.float32)
    h2 = lax.dot_general(x, wu_ref[0], dn, preferred_element_type=jnp.float32)
    h = (h1 * jax.nn.sigmoid(h1)) * h2
    out_ref[pl.ds(base, TILE), :] = lax.dot_general(
        h, wd_ref[0], dn, preferred_element_type=jnp.float32)


def _grouped_ffn(x_rows, Wg, Wu, Wd, tile_e, tile_start, nt):
    p, d = x_rows.shape
    _, i, _ = Wg.shape
    grid_spec = pltpu.PrefetchScalarGridSpec(
        num_scalar_prefetch=2,
        grid=(nt,),
        in_specs=[
            pl.BlockSpec((p, d), lambda t, e, b: (0, 0)),
            pl.BlockSpec((1, i, d), lambda t, e, b: (e[t], 0, 0)),
            pl.BlockSpec((1, i, d), lambda t, e, b: (e[t], 0, 0)),
            pl.BlockSpec((1, d, i), lambda t, e, b: (e[t], 0, 0)),
        ],
        out_specs=pl.BlockSpec((p, d), lambda t, e, b: (0, 0)),
    )
    return pl.pallas_call(
        _ffn_body,
        grid_spec=grid_spec,
        out_shape=jax.ShapeDtypeStruct((p, d), jnp.float32),
        compiler_params=pltpu.CompilerParams(
            dimension_semantics=("arbitrary",)),
    )(tile_e, tile_start, x_rows, Wg, Wu, Wd)


def _sc_gather_rows(table, idx):
    _, d = table.shape
    n = idx.shape[0]
    b_per_w = n // _NW
    n_chunks = 1
    while (b_per_w // n_chunks) * d * 4 > _SC_BUF_BYTES:
        n_chunks *= 2
    chunk = b_per_w // n_chunks
    mesh = plsc.VectorSubcoreMesh(
        core_axis_name="c", subcore_axis_name="s",
        num_cores=_SC_CORES, num_subcores=_SC_SUBCORES)

    @functools.partial(
        pl.kernel,
        out_type=jax.ShapeDtypeStruct((n, d), jnp.float32),
        mesh=mesh,
        scratch_types=[
            pltpu.VMEM((chunk,), jnp.int32),
            pltpu.VMEM((chunk, d), jnp.float32),
            pltpu.SemaphoreType.DMA,
        ],
    )
    def k(table_hbm, idx_hbm, out_hbm, idx_v, rows_v, sem):
        wid = lax.axis_index("s") * _SC_CORES + lax.axis_index("c")
        base = wid * b_per_w
        for c in range(n_chunks):
            off = base + c * chunk
            pltpu.sync_copy(idx_hbm.at[pl.ds(off, chunk)], idx_v)
            pltpu.async_copy(table_hbm.at[idx_v], rows_v, sem).wait()
            pltpu.sync_copy(rows_v, out_hbm.at[pl.ds(off, chunk)])

    return k(table, idx)


def kernel(hidden_states, gate_w, Wg, Wu, Wd):
    bsz, seq_len, d = hidden_states.shape
    e = gate_w.shape[0]
    flat = hidden_states.reshape(-1, d)
    s = flat.shape[0]
    p = s + PAD_ROWS

    logits, eids = _router(flat, gate_w)
    eid = eids[:, 0]

    sorted_eid, perm = lax.sort((eid, jnp.arange(s, dtype=jnp.int32)),
                                dimension=0, num_keys=1, is_stable=True)
    counts = jnp.bincount(eid, length=e).astype(jnp.int32)
    starts = jnp.concatenate(
        [jnp.zeros((1,), jnp.int32), jnp.cumsum(counts)[:-1].astype(jnp.int32)])
    acounts = (counts + 7) & ~7
    astarts = jnp.concatenate(
        [jnp.zeros((1,), jnp.int32),
         jnp.cumsum(acounts)[:-1].astype(jnp.int32)])

    tiles_per_e = (counts + TILE - 1) // TILE
    nt = s // TILE + e
    tile_e = jnp.repeat(jnp.arange(e, dtype=jnp.int32), tiles_per_e,
                        total_repeat_length=nt)
    tile_base = jnp.concatenate(
        [jnp.zeros((1,), jnp.int32),
         jnp.cumsum(tiles_per_e)[:-1].astype(jnp.int32)])
    tile_k = jnp.arange(nt, dtype=jnp.int32) - tile_base[tile_e]
    tile_start = jnp.clip(astarts[tile_e] + tile_k * TILE, 0, p - TILE)

    sorted_pos = jnp.arange(s, dtype=jnp.int32)
    slot_of_sorted = astarts[sorted_eid] + (sorted_pos - starts[sorted_eid])
    slot_token = jnp.zeros((p,), jnp.int32).at[slot_of_sorted].set(perm)
    token_slot = jnp.zeros((s,), jnp.int32).at[perm].set(slot_of_sorted)

    x_rows = _sc_gather_rows(flat, slot_token)
    out_rows = _grouped_ffn(x_rows, Wg, Wu, Wd, tile_e, tile_start, nt)
    out_flat = _sc_gather_rows(out_rows, token_slot)

    return out_flat.reshape(bsz, seq_len, d), logits
